# Initial kernel scaffold; baseline (speedup 1.0000x reference)
import jax
import jax.numpy as jnp
from jax.experimental import pallas as pl


def kernel(input, adj, W, a):
    raise NotImplementedError("write your pallas kernel here")


# speedup vs baseline: 23.3922x; 23.3922x over previous
import functools

import jax
import jax.numpy as jnp
from jax import lax
from jax.experimental import pallas as pl
from jax.experimental.pallas import tpu as pltpu
from jax.experimental.pallas import tpu_sc as plsc

ALPHA = 0.2
F = 128
WIDE = 144
NP = 10240
BLK = 512
NT = 16
K = 80


def _stage1_body(x_ref, w_ref, a_ref, h_ref, s1_ref, s2_ref):
    xb = x_ref[0]
    h = jnp.dot(xb, w_ref[...], preferred_element_type=jnp.float32)
    a1 = a_ref[0, :F]
    a2 = a_ref[0, F:]
    s1 = jnp.sum(h * a1[None, :], axis=1)
    s2 = jnp.sum(h * a2[None, :], axis=1)
    lane = lax.broadcasted_iota(jnp.int32, (BLK, WIDE - F), 1)
    pad = jnp.where(lane == 0, 1.0, 0.0).astype(jnp.float32)
    h_ref[0] = jnp.concatenate([h, pad], axis=1)
    s1_ref[0, 0] = s1
    s2_ref[0, 0] = s2


def _stage3_body(acc_ref, o_ref):
    blk = acc_ref[0]
    hp = blk[:, :F]
    rs = blk[:, F:F + 1]
    v = hp / (rs + 0.001)
    o_ref[0] = jnp.where(v > 0, v, jnp.exp(jnp.minimum(v, 0.0)) - 1.0)


def _make_sc_kernel(B, E):
    ept = E // NT
    nch = ept // K
    rpt = NP // NT
    mesh = plsc.VectorSubcoreMesh(core_axis_name="c", subcore_axis_name="s")

    @functools.partial(
        pl.kernel,
        out_type=jax.ShapeDtypeStruct((B, NP, WIDE), jnp.float32),
        mesh=mesh,
        compiler_params=pltpu.CompilerParams(
            needs_layout_passes=False, use_tc_tiling_on_sc=False),
        scratch_types=[
            pltpu.VMEM((NP,), jnp.float32),
            pltpu.VMEM((NP,), jnp.float32),
            pltpu.VMEM((2, K), jnp.int32),
            pltpu.VMEM((K, WIDE), jnp.float32),
            pltpu.VMEM((K,), jnp.float32),
            pltpu.VMEM_SHARED((NP, WIDE), jnp.float32),
        ],
    )
    def sc_kernel(h_hbm, s1_hbm, s2_hbm, row_hbm, col_hbm, zeros_hbm, out_hbm,
                  s1_ref, s2_ref, idx_ref, gbuf, ee_ref, acc):
        c = lax.axis_index("c")
        t = lax.axis_index("s")
        row0 = t * rpt
        pltpu.sync_copy(zeros_hbm.at[pl.ds(row0, rpt)],
                        acc.at[pl.ds(row0, rpt)])
        pltpu.sync_copy(s1_hbm.at[c, 0], s1_ref)
        pltpu.sync_copy(s2_hbm.at[c, 0], s2_ref)
        plsc.subcore_barrier()

        base0 = c * E + t * ept

        @pl.loop(0, nch)
        def _(k):
            base = base0 + k * K
            pltpu.sync_copy(row_hbm.at[pl.ds(base, K)], idx_ref.at[0])
            pltpu.sync_copy(col_hbm.at[pl.ds(base, K)], idx_ref.at[1])
            pltpu.sync_copy(h_hbm.at[c].at[idx_ref.at[1]], gbuf)
            for j in range(K // 16):
                sl = pl.ds(j * 16, 16)
                ridx = idx_ref[0, sl]
                cidx = idx_ref[1, sl]
                s1r = plsc.load_gather(s1_ref, [ridx])
                s2c = plsc.load_gather(s2_ref, [cidx])
                s = s1r + s2c
                lr = jnp.where(s > 0, s, ALPHA * s)
                ee_ref[sl] = jnp.exp(-lr)

            @pl.loop(0, K)
            def _(r):
                e = plsc.load_gather(ee_ref, [lax.broadcast(r, (16,))])
                for q in range(WIDE // 16):
                    slq = pl.ds(q * 16, 16)
                    gbuf[r, slq] = gbuf[r, slq] * e

            pltpu.sync_copy(gbuf, acc.at[idx_ref.at[0]], add=True)

        plsc.subcore_barrier()
        pltpu.sync_copy(acc.at[pl.ds(row0, rpt)],
                        out_hbm.at[c].at[pl.ds(row0, rpt)])

    return sc_kernel


def kernel(input, adj, W, a):
    B, N, _ = input.shape
    E = adj.shape[2]
    x = jnp.pad(input, ((0, 0), (0, NP - N), (0, 0)))

    grid = (B, NP // BLK)
    h_aug, s1, s2 = pl.pallas_call(
        _stage1_body,
        grid=grid,
        in_specs=[
            pl.BlockSpec((1, BLK, F), lambda b, i: (b, i, 0)),
            pl.BlockSpec((F, F), lambda b, i: (0, 0)),
            pl.BlockSpec((1, 2 * F), lambda b, i: (0, 0)),
        ],
        out_specs=[
            pl.BlockSpec((1, BLK, WIDE), lambda b, i: (b, i, 0)),
            pl.BlockSpec((1, 1, BLK), lambda b, i: (b, 0, i)),
            pl.BlockSpec((1, 1, BLK), lambda b, i: (b, 0, i)),
        ],
        out_shape=[
            jax.ShapeDtypeStruct((B, NP, WIDE), jnp.float32),
            jax.ShapeDtypeStruct((B, 1, NP), jnp.float32),
            jax.ShapeDtypeStruct((B, 1, NP), jnp.float32),
        ],
    )(x, W, a)

    zeros = jnp.zeros((NP, WIDE), dtype=jnp.float32)
    row_flat = adj[0].reshape(B * E)
    col_flat = adj[1].reshape(B * E)
    sc_kernel = _make_sc_kernel(B, E)
    acc = sc_kernel(h_aug, s1, s2, row_flat, col_flat, zeros)

    out = pl.pallas_call(
        _stage3_body,
        grid=grid,
        in_specs=[pl.BlockSpec((1, BLK, WIDE), lambda b, i: (b, i, 0))],
        out_specs=pl.BlockSpec((1, BLK, F), lambda b, i: (b, i, 0)),
        out_shape=jax.ShapeDtypeStruct((B, NP, F), jnp.float32),
    )(acc)

    return out[:, :N, :]


# speedup vs baseline: 33.8864x; 1.4486x over previous
import functools

import jax
import jax.numpy as jnp
from jax import lax
from jax.experimental import pallas as pl
from jax.experimental.pallas import tpu as pltpu
from jax.experimental.pallas import tpu_sc as plsc

ALPHA = 0.2
F = 128
WIDE = 144
NP = 10240
BLK = 512
NT = 16
K = 80


def _stage1_body(x_ref, w_ref, a_ref, h_ref, s1_ref):
    xb = x_ref[0]
    h = jnp.dot(xb, w_ref[...], preferred_element_type=jnp.float32)
    a1 = a_ref[0, :F]
    a2 = a_ref[0, F:]
    s1 = jnp.sum(h * a1[None, :], axis=1, keepdims=True)
    s2 = jnp.sum(h * a2[None, :], axis=1, keepdims=True)
    z15 = jnp.zeros((BLK, 15), jnp.float32)
    h_ref[0] = jnp.concatenate([h, s2, z15], axis=1)
    s1_ref[0] = jnp.concatenate([s1, z15], axis=1)


def _stage3_body(acc_ref, o_ref):
    blk = acc_ref[0]
    hp = blk[:, :F]
    rs = blk[:, F:F + 1]
    v = hp / (rs + 0.001)
    o_ref[0] = jnp.where(v > 0, v, jnp.exp(jnp.minimum(v, 0.0)) - 1.0)


def _make_sc_kernel(B, nch, nch_real):
    rpt = NP // NT
    mesh = plsc.VectorSubcoreMesh(core_axis_name="c", subcore_axis_name="s")

    @functools.partial(
        pl.kernel,
        out_type=jax.ShapeDtypeStruct((B, NP, WIDE), jnp.float32),
        mesh=mesh,
        compiler_params=pltpu.CompilerParams(
            needs_layout_passes=False, use_tc_tiling_on_sc=False),
        scratch_types=[
            pltpu.VMEM((4, 2, K), jnp.int32),
            pltpu.VMEM((2, K, WIDE), jnp.float32),
            pltpu.VMEM((2, K, 16), jnp.float32),
            pltpu.VMEM((K,), jnp.float32),
            pltpu.VMEM_SHARED((NP, WIDE), jnp.float32),
            pltpu.SemaphoreType.DMA,
            pltpu.SemaphoreType.DMA,
            pltpu.SemaphoreType.DMA,
            pltpu.SemaphoreType.DMA,
            pltpu.SemaphoreType.DMA,
            pltpu.SemaphoreType.DMA,
        ],
    )
    def sc_kernel(h_hbm, s1_hbm, adjr_hbm, zh_hbm, out_hbm,
                  idx, gbuf, s1buf, ee_ref, acc,
                  gsem0, gsem1, isem0, isem1, ssem0, ssem1):
        c = lax.axis_index("c")
        t = lax.axis_index("s")
        row0 = t * rpt
        pltpu.sync_copy(zh_hbm.at[pl.ds(row0, rpt)],
                        acc.at[pl.ds(row0, rpt)])
        plsc.subcore_barrier()

        gsem = (gsem0, gsem1)
        isem = (isem0, isem1)
        ssem = (ssem0, ssem1)
        h_c = h_hbm.at[c]
        s1_c = s1_hbm.at[c]
        adj_ct = adjr_hbm.at[c, t]
        iota = lax.broadcasted_iota(jnp.int32, (16,), 0)
        c128 = jnp.full((16,), F, jnp.int32)

        def start_gathers(b2, b4):
            pltpu.async_copy(h_c.at[idx.at[b4, 1]], gbuf.at[b2], gsem[b2])
            pltpu.async_copy(s1_c.at[idx.at[b4, 0]], s1buf.at[b2], gsem[b2])

        def wait_gathers(b2, b4):
            pltpu.make_async_copy(h_c.at[idx.at[b4, 1]], gbuf.at[b2],
                                  gsem[b2]).wait()
            pltpu.make_async_copy(s1_c.at[idx.at[b4, 0]], s1buf.at[b2],
                                  gsem[b2]).wait()

        def compute(k, b2):
            gb = gbuf.at[b2]
            sb = s1buf.at[b2]
            m = jnp.where(k < nch_real, jnp.float32(1), jnp.float32(0))
            mv = lax.broadcast(m, (16,))
            for j in range(K // 16):
                rows = iota + j * 16
                s = (plsc.load_gather(sb, [rows, iota * 0])
                     + plsc.load_gather(gb, [rows, c128]))
                lr = jnp.where(s > 0, s, ALPHA * s)
                eg = jnp.exp(-lr) * mv
                ee_ref[pl.ds(j * 16, 16)] = eg
                plsc.store_scatter(gb, [rows, c128], eg)

            @pl.loop(0, K)
            def _(r):
                e = plsc.load_gather(ee_ref, [lax.broadcast(r, (16,))])
                for q in range(F // 16):
                    slq = pl.ds(q * 16, 16)
                    gb[r, slq] = gb[r, slq] * e

        def chunk(k, b):
            b2 = b % 2
            b2n = (b + 1) % 2
            b4 = b
            b4n = (b + 1) % 4
            b4nn = (b + 2) % 4
            b4p = (b + 3) % 4

            @pl.when(k > 0)
            def _():
                pltpu.make_async_copy(gbuf.at[b2n], acc.at[idx.at[b4p, 0]],
                                      ssem[b2n]).wait()

            @pl.when(k + 2 < nch)
            def _():
                pltpu.async_copy(adj_ct.at[k + 2], idx.at[b4nn], isem[b2])

            @pl.when(k + 1 < nch)
            def _():
                pltpu.make_async_copy(adj_ct.at[k + 1], idx.at[b4n],
                                      isem[b2n]).wait()
                start_gathers(b2n, b4n)

            wait_gathers(b2, b4)
            compute(k, b2)
            pltpu.async_copy(gbuf.at[b2], acc.at[idx.at[b4, 0]], ssem[b2],
                             add=True)

        pltpu.sync_copy(adj_ct.at[0], idx.at[0])
        pltpu.async_copy(adj_ct.at[1], idx.at[1], isem1)
        start_gathers(0, 0)

        @pl.loop(0, nch // 4)
        def _(g):
            k0 = g * 4
            for b in range(4):
                chunk(k0 + b, b)

        pltpu.make_async_copy(gbuf.at[1], acc.at[idx.at[3, 0]],
                              ssem[1]).wait()
        plsc.subcore_barrier()
        pltpu.sync_copy(acc.at[pl.ds(row0, rpt)],
                        out_hbm.at[c].at[pl.ds(row0, rpt)])

    return sc_kernel


def kernel(input, adj, W, a):
    B, N, _ = input.shape
    E = adj.shape[2]
    ept = E // NT
    nch_real = ept // K
    nch = ((nch_real + 3) // 4) * 4
    x = jnp.pad(input, ((0, 0), (0, NP - N), (0, 0)))

    grid = (B, NP // BLK)
    h_aug, s1w = pl.pallas_call(
        _stage1_body,
        grid=grid,
        in_specs=[
            pl.BlockSpec((1, BLK, F), lambda b, i: (b, i, 0)),
            pl.BlockSpec((F, F), lambda b, i: (0, 0)),
            pl.BlockSpec((1, 2 * F), lambda b, i: (0, 0)),
        ],
        out_specs=[
            pl.BlockSpec((1, BLK, WIDE), lambda b, i: (b, i, 0)),
            pl.BlockSpec((1, BLK, 16), lambda b, i: (b, i, 0)),
        ],
        out_shape=[
            jax.ShapeDtypeStruct((B, NP, WIDE), jnp.float32),
            jax.ShapeDtypeStruct((B, NP, 16), jnp.float32),
        ],
    )(x, W, a)

    row4 = adj[0].reshape(B, NT, nch_real, K)
    col4 = adj[1].reshape(B, NT, nch_real, K)
    pad = ((0, 0), (0, 0), (0, nch - nch_real), (0, 0))
    adjr = jnp.stack([jnp.pad(row4, pad), jnp.pad(col4, pad)], axis=3)
    zh = jnp.zeros((NP, WIDE), dtype=jnp.float32)
    sc_kernel = _make_sc_kernel(B, nch, nch_real)
    acc = sc_kernel(h_aug, s1w, adjr, zh)

    out = pl.pallas_call(
        _stage3_body,
        grid=grid,
        in_specs=[pl.BlockSpec((1, BLK, WIDE), lambda b, i: (b, i, 0))],
        out_specs=pl.BlockSpec((1, BLK, F), lambda b, i: (b, i, 0)),
        out_shape=jax.ShapeDtypeStruct((B, NP, F), jnp.float32),
    )(acc)

    return out[:, :N, :]


# speedup vs baseline: 39.6948x; 1.1714x over previous
import functools

import jax
import jax.numpy as jnp
from jax import lax
from jax.experimental import pallas as pl
from jax.experimental.pallas import tpu as pltpu
from jax.experimental.pallas import tpu_sc as plsc

ALPHA = 0.2
F = 128
WIDE = 144
NP = 10240
BLK = 512
NT = 16
K = 80
UNROLL = 6


def _stage1_body(x_ref, w_ref, a_ref, h_ref, s1_ref):
    xb = x_ref[0]
    h = jnp.dot(xb, w_ref[...], preferred_element_type=jnp.float32)
    a1 = a_ref[0, :F]
    a2 = a_ref[0, F:]
    s1 = jnp.sum(h * a1[None, :], axis=1, keepdims=True)
    s2 = jnp.sum(h * a2[None, :], axis=1, keepdims=True)
    z15 = jnp.zeros((BLK, 15), jnp.float32)
    h_ref[0] = jnp.concatenate([h, s2, z15], axis=1)
    s1_ref[0] = jnp.concatenate([s1, z15], axis=1)


def _stage3_body(acc_ref, o_ref):
    blk = acc_ref[0]
    hp = blk[:, :F]
    rs = blk[:, F:F + 1]
    v = hp / (rs + 0.001)
    o_ref[0] = jnp.where(v > 0, v, jnp.exp(jnp.minimum(v, 0.0)) - 1.0)


def _make_sc_kernel(B, nch):
    rpt = NP // NT
    mesh = plsc.VectorSubcoreMesh(core_axis_name="c", subcore_axis_name="s")

    @functools.partial(
        pl.kernel,
        out_type=jax.ShapeDtypeStruct((B, NP, WIDE), jnp.float32),
        mesh=mesh,
        compiler_params=pltpu.CompilerParams(
            needs_layout_passes=False, use_tc_tiling_on_sc=False),
        scratch_types=[
            pltpu.VMEM((UNROLL, 2, K), jnp.int32),
            pltpu.VMEM((3, K, WIDE), jnp.float32),
            pltpu.VMEM((2, K, 16), jnp.float32),
            pltpu.VMEM_SHARED((NP, WIDE), jnp.float32),
            pltpu.SemaphoreType.DMA,
            pltpu.SemaphoreType.DMA,
            pltpu.SemaphoreType.DMA,
            pltpu.SemaphoreType.DMA,
            pltpu.SemaphoreType.DMA,
            pltpu.SemaphoreType.DMA,
            pltpu.SemaphoreType.DMA,
            pltpu.SemaphoreType.DMA,
        ],
    )
    def sc_kernel(h_hbm, s1_hbm, adjr_hbm, zh_hbm, out_hbm,
                  idx, gbuf, s1buf, acc,
                  gsem0, gsem1, gsem2, isem0, isem1, ssem0, ssem1, ssem2):
        c = lax.axis_index("c")
        t = lax.axis_index("s")
        row0 = t * rpt
        pltpu.sync_copy(zh_hbm.at[pl.ds(row0, rpt)],
                        acc.at[pl.ds(row0, rpt)])
        plsc.subcore_barrier()

        gsem = (gsem0, gsem1, gsem2)
        isem = (isem0, isem1)
        ssem = (ssem0, ssem1, ssem2)
        h_c = h_hbm.at[c]
        s1_c = s1_hbm.at[c]
        adj_ct = adjr_hbm.at[c, t]
        iota = lax.broadcasted_iota(jnp.int32, (16,), 0)
        zlane = iota * 0
        c128 = jnp.full((16,), F, jnp.int32)

        def start_gathers(b3, b2, b6):
            pltpu.async_copy(h_c.at[idx.at[b6, 1]], gbuf.at[b3], gsem[b3])
            pltpu.async_copy(s1_c.at[idx.at[b6, 0]], s1buf.at[b2], gsem[b3])

        def wait_gathers(b3, b2, b6):
            pltpu.make_async_copy(h_c.at[idx.at[b6, 1]], gbuf.at[b3],
                                  gsem[b3]).wait()
            pltpu.make_async_copy(s1_c.at[idx.at[b6, 0]], s1buf.at[b2],
                                  gsem[b3]).wait()

        def wait_scatter(b3, b6):
            pltpu.make_async_copy(gbuf.at[b3], acc.at[idx.at[b6, 0]],
                                  ssem[b3]).wait()

        def compute(b3, b2):
            gb = gbuf.at[b3]
            sb = s1buf.at[b2]
            for j in range(K // 16):
                rows = iota + j * 16
                s = (plsc.load_gather(sb, [rows, zlane])
                     + plsc.load_gather(gb, [rows, c128]))
                lr = jnp.where(s > 0, s, ALPHA * s)
                plsc.store_scatter(gb, [rows, c128], jnp.exp(-lr))

            @plsc.parallel_loop(0, K, unroll=2)
            def _(r):
                e = plsc.load_gather(gb, [lax.broadcast(r, (16,)), c128])
                for q in range(F // 16):
                    slq = pl.ds(q * 16, 16)
                    gb[r, slq] = gb[r, slq] * e

        def chunk(k, b):
            b3 = b % 3
            b3n = (b + 1) % 3
            b2 = b % 2
            b2n = (b + 1) % 2
            b6 = b
            b6n = (b + 1) % UNROLL
            b6nn = (b + 2) % UNROLL
            b6pp = (b + UNROLL - 2) % UNROLL

            @pl.when(k > 1)
            def _():
                wait_scatter(b3n, b6pp)

            @pl.when(k + 2 < nch)
            def _():
                pltpu.async_copy(adj_ct.at[k + 2], idx.at[b6nn], isem[b2])

            @pl.when(k + 1 < nch)
            def _():
                pltpu.make_async_copy(adj_ct.at[k + 1], idx.at[b6n],
                                      isem[b2n]).wait()
                start_gathers(b3n, b2n, b6n)

            wait_gathers(b3, b2, b6)
            compute(b3, b2)
            pltpu.async_copy(gbuf.at[b3], acc.at[idx.at[b6, 0]], ssem[b3],
                             add=True)

        pltpu.sync_copy(adj_ct.at[0], idx.at[0])
        pltpu.async_copy(adj_ct.at[1], idx.at[1], isem1)
        start_gathers(0, 0, 0)

        @pl.loop(0, nch // UNROLL)
        def _(g):
            k0 = g * UNROLL
            for b in range(UNROLL):
                chunk(k0 + b, b)

        wait_scatter((nch - 2) % 3, (nch - 2) % UNROLL)
        wait_scatter((nch - 1) % 3, (nch - 1) % UNROLL)
        plsc.subcore_barrier()
        pltpu.sync_copy(acc.at[pl.ds(row0, rpt)],
                        out_hbm.at[c].at[pl.ds(row0, rpt)])

    return sc_kernel


def kernel(input, adj, W, a):
    B, N, _ = input.shape
    E = adj.shape[2]
    ept = E // NT
    nchu = (ept + K - 1) // K
    nch = (nchu + UNROLL - 1) // UNROLL * UNROLL
    ept_pad = nch * K
    x = jnp.pad(input, ((0, 0), (0, NP - N), (0, 0)))

    grid = (B, NP // BLK)
    h_aug, s1w = pl.pallas_call(
        _stage1_body,
        grid=grid,
        in_specs=[
            pl.BlockSpec((1, BLK, F), lambda b, i: (b, i, 0)),
            pl.BlockSpec((F, F), lambda b, i: (0, 0)),
            pl.BlockSpec((1, 2 * F), lambda b, i: (0, 0)),
        ],
        out_specs=[
            pl.BlockSpec((1, BLK, WIDE), lambda b, i: (b, i, 0)),
            pl.BlockSpec((1, BLK, 16), lambda b, i: (b, i, 0)),
        ],
        out_shape=[
            jax.ShapeDtypeStruct((B, NP, WIDE), jnp.float32),
            jax.ShapeDtypeStruct((B, NP, 16), jnp.float32),
        ],
    )(x, W, a)

    pad = ((0, 0), (0, 0), (0, ept_pad - ept))
    row3 = jnp.pad(adj[0].reshape(B, NT, ept), pad, constant_values=NP - 1)
    col3 = jnp.pad(adj[1].reshape(B, NT, ept), pad, constant_values=NP - 1)
    adjr = jnp.stack([row3.reshape(B, NT, nch, K),
                      col3.reshape(B, NT, nch, K)], axis=3)
    zh = jnp.zeros((NP, WIDE), dtype=jnp.float32)
    sc_kernel = _make_sc_kernel(B, nch)
    acc = sc_kernel(h_aug, s1w, adjr, zh)

    out = pl.pallas_call(
        _stage3_body,
        grid=grid,
        in_specs=[pl.BlockSpec((1, BLK, WIDE), lambda b, i: (b, i, 0))],
        out_specs=pl.BlockSpec((1, BLK, F), lambda b, i: (b, i, 0)),
        out_shape=jax.ShapeDtypeStruct((B, NP, F), jnp.float32),
    )(acc)

    return out[:, :N, :]
